# baseline (speedup 1.0000x reference)
import functools

import jax
import jax.numpy as jnp
from jax import lax
from jax.experimental import pallas as pl
from jax.experimental.pallas import tpu as pltpu
from jax.experimental.pallas import tpu_sc as plsc

T, V, D, L = 16384, 100000, 128, 64


def _make_sc_gather(B, d):
    NC, NS = 2, 16
    NW = NC * NS
    b_per_w = B // NW
    n_chunks = b_per_w // 128
    mesh = plsc.VectorSubcoreMesh(core_axis_name="c", subcore_axis_name="s")

    def body(idx_hbm, table_hbm, out_hbm, idx_v, rows_v, sem):
        wid = lax.axis_index("s") * NC + lax.axis_index("c")
        base = wid * b_per_w
        pltpu.sync_copy(idx_hbm.at[wid], idx_v)
        copies = []
        for j in range(n_chunks):
            copies.append(
                pltpu.async_copy(
                    table_hbm.at[idx_v.at[j]],
                    rows_v.at[pl.ds(j * 128, 128)],
                    sem,
                )
            )
        for c in copies:
            c.wait()
        pltpu.sync_copy(rows_v, out_hbm.at[pl.ds(base, b_per_w)])

    return pl.kernel(
        body,
        out_type=jax.ShapeDtypeStruct((B, d), jnp.float32),
        mesh=mesh,
        scratch_types=[
            pltpu.VMEM((n_chunks, 128), jnp.int32),
            pltpu.VMEM((b_per_w, d), jnp.float32),
            pltpu.SemaphoreType.DMA,
        ],
    ), NW, n_chunks


_sc_gather, _NW, _NCHUNKS = _make_sc_gather(T, D)


def _mm_body(g_ref, wt_ref, b0_ref, b1_ref, out_ref):
    acc = jnp.dot(g_ref[...], wt_ref[...], preferred_element_type=jnp.float32)
    out_ref[...] = acc + b0_ref[...] + b1_ref[...]


def _tc_matmul(g, wt, b0, b1):
    bT = 2048
    return pl.pallas_call(
        _mm_body,
        out_shape=jax.ShapeDtypeStruct((T, D), jnp.float32),
        grid=(T // bT,),
        in_specs=[
            pl.BlockSpec((bT, 2 * L), lambda i: (i, 0)),
            pl.BlockSpec((2 * L, D), lambda i: (0, 0)),
            pl.BlockSpec((1, D), lambda i: (0, 0)),
            pl.BlockSpec((1, D), lambda i: (0, 0)),
        ],
        out_specs=pl.BlockSpec((bT, D), lambda i: (i, 0)),
    )(g, wt, b0, b1)


def _add_body(a_ref, b_ref, out_ref):
    out_ref[...] = a_ref[...] + b_ref[...]


def _tc_add(a, b):
    bT = 2048
    return pl.pallas_call(
        _add_body,
        out_shape=jax.ShapeDtypeStruct((T, D), jnp.float32),
        grid=(T // bT,),
        in_specs=[
            pl.BlockSpec((bT, D), lambda i: (i, 0)),
            pl.BlockSpec((bT, D), lambda i: (i, 0)),
        ],
        out_specs=pl.BlockSpec((bT, D), lambda i: (i, 0)),
    )(a, b)


def kernel(sentence_data, batch_sizes, gazetteers_data, word_table, W0, b0, W1, b1):
    del batch_sizes
    idx = sentence_data.reshape(_NW, _NCHUNKS, 128)
    wemb = _sc_gather(idx, word_table)
    wt = jnp.concatenate([W0.T, W1.T], axis=0)
    gaz = _tc_matmul(gazetteers_data, wt, b0[None, :], b1[None, :])
    return _tc_add(wemb, gaz)


# speedup vs baseline: 1.1543x; 1.1543x over previous
import functools

import jax
import jax.numpy as jnp
from jax import lax
from jax.experimental import pallas as pl
from jax.experimental.pallas import tpu as pltpu
from jax.experimental.pallas import tpu_sc as plsc

T, V, D, L = 16384, 100000, 128, 64


def _make_sc_gather(B, d):
    NC, NS = 2, 16
    NW = NC * NS
    b_per_w = B // NW
    n_chunks = b_per_w // 128
    mesh = plsc.VectorSubcoreMesh(core_axis_name="c", subcore_axis_name="s")

    def body(idx_hbm, table_hbm, out_hbm, idx_v, rows_v, sem):
        wid = lax.axis_index("s") * NC + lax.axis_index("c")
        base = wid * b_per_w
        pltpu.sync_copy(idx_hbm.at[wid], idx_v)
        copies = []
        for j in range(n_chunks):
            copies.append(
                pltpu.async_copy(
                    table_hbm.at[idx_v.at[j]],
                    rows_v.at[pl.ds(j * 128, 128)],
                    sem,
                )
            )
        for c in copies:
            c.wait()
        pltpu.sync_copy(rows_v, out_hbm.at[pl.ds(base, b_per_w)])

    return pl.kernel(
        body,
        out_type=jax.ShapeDtypeStruct((B, d), jnp.float32),
        mesh=mesh,
        scratch_types=[
            pltpu.VMEM((n_chunks, 128), jnp.int32),
            pltpu.VMEM((b_per_w, d), jnp.float32),
            pltpu.SemaphoreType.DMA,
        ],
    ), NW, n_chunks


_sc_gather, _NW, _NCHUNKS = _make_sc_gather(T, D)


def _tc_body(g_ref, wemb_ref, wt_ref, b0_ref, b1_ref, out_ref):
    acc = jnp.dot(g_ref[...], wt_ref[...], preferred_element_type=jnp.float32)
    out_ref[...] = wemb_ref[...] + acc + b0_ref[...] + b1_ref[...]


def _tc_matmul_add(g, wemb, wt, b0, b1):
    bT = 4096
    return pl.pallas_call(
        _tc_body,
        out_shape=jax.ShapeDtypeStruct((T, D), jnp.float32),
        grid=(T // bT,),
        in_specs=[
            pl.BlockSpec((bT, 2 * L), lambda i: (i, 0)),
            pl.BlockSpec((bT, D), lambda i: (i, 0)),
            pl.BlockSpec((2 * L, D), lambda i: (0, 0)),
            pl.BlockSpec((1, D), lambda i: (0, 0)),
            pl.BlockSpec((1, D), lambda i: (0, 0)),
        ],
        out_specs=pl.BlockSpec((bT, D), lambda i: (i, 0)),
    )(g, wemb, wt, b0, b1)


def kernel(sentence_data, batch_sizes, gazetteers_data, word_table, W0, b0, W1, b1):
    del batch_sizes
    idx = sentence_data.reshape(_NW, _NCHUNKS, 128)
    wemb = _sc_gather(idx, word_table)
    wt = jnp.concatenate([W0.T, W1.T], axis=0)
    return _tc_matmul_add(gazetteers_data, wemb, wt, b0[None, :], b1[None, :])
